# baseline (speedup 1.0000x reference)
import jax
import jax.numpy as jnp
from jax.experimental import pallas as pl
from jax.experimental.pallas import tpu as pltpu

_NUM_EXPERTS = 16
_TOKEN_BLOCK = 2048


def _wsum_body(w_ref, b_ref, s_ref):
    d = w_ref.shape[1]
    s_ref[0, 0, :d] = jnp.sum(w_ref[0], axis=1)
    s_ref[0, 0, d:] = b_ref[0, 0, :]


def _moe_body(x_ref, gwt_ref, gb_ref, sb_ref, o_ref):
    x = x_ref[...]
    logits = jax.lax.dot_general(
        x, gwt_ref[...], (((1,), (1,)), ((), ())),
        preferred_element_type=jnp.float32,
    ) + gb_ref[...]
    g = jax.nn.softmax(logits, axis=-1)
    e_ids = jax.lax.broadcasted_iota(jnp.int32, g.shape, 1)
    i1 = jnp.argmax(g, axis=-1)
    oh1 = e_ids == i1[:, None]
    i2 = jnp.argmax(jnp.where(oh1, -1.0, g), axis=-1)
    oh2 = e_ids == i2[:, None]
    p = jnp.where(oh1 | oh2, g, 0.0)
    d = x.shape[1]
    q = jax.lax.dot_general(
        p, sb_ref[...], (((1,), (0,)), ((), ())),
        preferred_element_type=jnp.float32,
    )
    o_ref[...] = x * q[:, :d] + q[:, d:]


def kernel(x, gate_weight, gate_bias, experts_weights, experts_bias):
    n_tokens, d_in = x.shape
    n_exp, _, d_out = experts_weights.shape

    sb = pl.pallas_call(
        _wsum_body,
        grid=(n_exp,),
        in_specs=[
            pl.BlockSpec((1, d_in, d_out), lambda e: (e, 0, 0)),
            pl.BlockSpec((1, 1, d_out), lambda e: (e, 0, 0)),
        ],
        out_specs=pl.BlockSpec((1, 1, d_in + d_out), lambda e: (e, 0, 0)),
        out_shape=jax.ShapeDtypeStruct((n_exp, 1, d_in + d_out), jnp.float32),
    )(experts_weights, experts_bias.reshape(n_exp, 1, d_out))
    sb = sb.reshape(n_exp, d_in + d_out)

    blk = _TOKEN_BLOCK
    out = pl.pallas_call(
        _moe_body,
        grid=(n_tokens // blk,),
        in_specs=[
            pl.BlockSpec((blk, d_in), lambda i: (i, 0)),
            pl.BlockSpec((n_exp, d_in), lambda i: (0, 0)),
            pl.BlockSpec((1, n_exp), lambda i: (0, 0)),
            pl.BlockSpec((n_exp, d_in + d_out), lambda i: (0, 0)),
        ],
        out_specs=pl.BlockSpec((blk, d_out), lambda i: (i, 0)),
        out_shape=jax.ShapeDtypeStruct((n_tokens, d_out), jnp.float32),
    )(x, gate_weight, gate_bias.reshape(1, n_exp), sb)
    return out


# speedup vs baseline: 1.0573x; 1.0573x over previous
import jax
import jax.numpy as jnp
from jax.experimental import pallas as pl
from jax.experimental.pallas import tpu as pltpu

_TOKEN_BLOCK = 2048


def _moe_body(n_exp, w_ref, b_ref, x_ref, gw_ref, gb_ref, o_ref, sb_scr):
    step = pl.program_id(0)
    d = x_ref.shape[1]

    @pl.when(step < n_exp)
    def _w_phase():
        sb_scr[pl.ds(step, 1), :d] = jnp.sum(w_ref[0], axis=1)[None, :]
        sb_scr[pl.ds(step, 1), d:] = b_ref[0]

    @pl.when(step >= n_exp)
    def _t_phase():
        x = x_ref[...]
        logits = jax.lax.dot_general(
            x, gw_ref[...], (((1,), (1,)), ((), ())),
            preferred_element_type=jnp.float32,
        ) + gb_ref[...]
        g = jax.nn.softmax(logits, axis=-1)
        e_ids = jax.lax.broadcasted_iota(jnp.int32, g.shape, 1)
        i1 = jnp.argmax(g, axis=-1)
        oh1 = e_ids == i1[:, None]
        i2 = jnp.argmax(jnp.where(oh1, -1.0, g), axis=-1)
        oh2 = e_ids == i2[:, None]
        p = jnp.where(oh1 | oh2, g, 0.0)
        q = jax.lax.dot_general(
            p, sb_scr[...], (((1,), (0,)), ((), ())),
            preferred_element_type=jnp.float32,
        )
        o_ref[...] = x * q[:, :d] + q[:, d:]


def kernel(x, gate_weight, gate_bias, experts_weights, experts_bias):
    n_tokens, d_in = x.shape
    n_exp, _, d_out = experts_weights.shape
    blk = _TOKEN_BLOCK
    n_tok_blocks = n_tokens // blk

    import functools
    body = functools.partial(_moe_body, n_exp)
    out = pl.pallas_call(
        body,
        grid=(n_exp + n_tok_blocks,),
        in_specs=[
            pl.BlockSpec((1, d_in, d_out),
                         lambda i: (jnp.minimum(i, n_exp - 1), 0, 0)),
            pl.BlockSpec((1, 1, d_out),
                         lambda i: (jnp.minimum(i, n_exp - 1), 0, 0)),
            pl.BlockSpec((blk, d_in),
                         lambda i: (jnp.maximum(i - n_exp, 0), 0)),
            pl.BlockSpec((n_exp, d_in), lambda i: (0, 0)),
            pl.BlockSpec((1, n_exp), lambda i: (0, 0)),
        ],
        out_specs=pl.BlockSpec((blk, d_out),
                               lambda i: (jnp.maximum(i - n_exp, 0), 0)),
        out_shape=jax.ShapeDtypeStruct((n_tokens, d_out), jnp.float32),
        scratch_shapes=[pltpu.VMEM((n_exp, d_in + d_out), jnp.float32)],
    )(experts_weights, experts_bias.reshape(n_exp, 1, d_out), x, gate_weight,
      gate_bias.reshape(1, n_exp))
    return out
